# baseline (speedup 1.0000x reference)
import functools

import jax
import jax.numpy as jnp
from jax import lax
from jax.experimental import pallas as pl
from jax.experimental.pallas import tpu as pltpu
from jax.experimental.pallas import tpu_sc as plsc

_B = 16384
_D = 64
_PRE = 384
_NC, _NS = 2, 16
_NW = _NC * _NS
_BPW = _B // _NW
_C = 64
_NCHUNK = _BPW // _C
_G = 4
_SG = _C // _G


def _sc_gather_body(h_idx, t_idx, c_idx,
                    inst_tab, conc_tab, rad_tab, cpre_tab,
                    h_out, t_out, c_out, rad_out, cpre_out,
                    ci_v, hi_v, ti_v, cpre_v,
                    gsem0, gsem1, rsem, wsem):
    wid = lax.axis_index("s") * _NC + lax.axis_index("c")
    base = wid * _BPW
    pltpu.sync_copy(c_idx.at[pl.ds(base, _BPW)], ci_v)
    pltpu.sync_copy(h_idx.at[pl.ds(base, _BPW)], hi_v)
    pltpu.sync_copy(t_idx.at[pl.ds(base, _BPW)], ti_v)

    def group_body(g, carry):
        vh = hi_v[pl.ds(g * 16, 16)]
        vt = ti_v[pl.ds(g * 16, 16)]
        vc = ci_v[pl.ds(g * 16, 16)]
        for l in range(16):
            o = pl.ds(base + g * 16 + l, 1)
            pltpu.async_copy(inst_tab.at[pl.ds(vh[l], 1)], h_out.at[o], rsem)
            pltpu.async_copy(inst_tab.at[pl.ds(vt[l], 1)], t_out.at[o], rsem)
            pltpu.async_copy(conc_tab.at[pl.ds(vc[l], 1)], c_out.at[o], rsem)
            pltpu.async_copy(rad_tab.at[pl.ds(vc[l], 1)], rad_out.at[o], rsem)
        return carry

    lax.fori_loop(0, _BPW // 16, group_body, 0)

    gsems = (gsem0, gsem1)

    def fire_gathers(k):
        s = k % 2
        return [pltpu.async_copy(
            cpre_tab.at[ci_v.at[pl.ds(k * _C + g * _SG, _SG)]],
            cpre_v.at[s].at[pl.ds(g * _SG, _SG)],
            gsems[s]) for g in range(_G)]

    gcps = {0: fire_gathers(0), 1: fire_gathers(1)}
    for k in range(_NCHUNK):
        for cp in gcps.pop(k):
            cp.wait()
        wcp = pltpu.async_copy(cpre_v.at[k % 2],
                               cpre_out.at[pl.ds(base + k * _C, _C)], wsem)
        wcp.wait()
        if k + 2 < _NCHUNK:
            gcps[k + 2] = fire_gathers(k + 2)

    gb = pl.ds(base, _BPW)
    pltpu.make_async_copy(inst_tab.at[pl.ds(0, _BPW)], h_out.at[gb],
                          rsem).wait()
    pltpu.make_async_copy(inst_tab.at[pl.ds(0, _BPW)], t_out.at[gb],
                          rsem).wait()
    pltpu.make_async_copy(conc_tab.at[pl.ds(0, _BPW)], c_out.at[gb],
                          rsem).wait()
    pltpu.make_async_copy(rad_tab.at[pl.ds(0, _BPW)], rad_out.at[gb],
                          rsem).wait()


_sc_gather = functools.partial(
    pl.kernel,
    out_type=(
        jax.ShapeDtypeStruct((_B, _D), jnp.float32),
        jax.ShapeDtypeStruct((_B, _D), jnp.float32),
        jax.ShapeDtypeStruct((_B, _D), jnp.float32),
        jax.ShapeDtypeStruct((_B, 1), jnp.float32),
        jax.ShapeDtypeStruct((_B, _PRE), jnp.float32),
    ),
    mesh=plsc.VectorSubcoreMesh(core_axis_name="c", subcore_axis_name="s",
                                num_cores=_NC, num_subcores=_NS),
    compiler_params=pltpu.CompilerParams(use_tc_tiling_on_sc=True),
    scratch_types=[
        pltpu.VMEM((_BPW,), jnp.int32),
        pltpu.VMEM((_BPW,), jnp.int32),
        pltpu.VMEM((_BPW,), jnp.int32),
        pltpu.VMEM((2, _C, _PRE), jnp.float32),
        pltpu.SemaphoreType.DMA,
        pltpu.SemaphoreType.DMA,
        pltpu.SemaphoreType.DMA,
        pltpu.SemaphoreType.DMA,
    ],
)(_sc_gather_body)


_TB = 1024


def _tc_body(h_ref, t_ref, c_ref, rad_ref, ridx_ref, cpre_ref,
             rel_ref, imap_ref, winT_ref, b_ref, st_ref, de_ref, di_ref):
    h = h_ref[...]
    rlane = lax.broadcasted_iota(jnp.int32, (_TB, 128), 1)
    ronehot = jnp.where(rlane == ridx_ref[...], 1.0, 0.0)
    r = jnp.dot(ronehot, rel_ref[...], preferred_element_type=jnp.float32)
    d = h + r - t_ref[...]
    st_ref[...] = jnp.sum(d * d, axis=1, keepdims=True)
    e = h - c_ref[...]
    rad = rad_ref[...]
    de_ref[...] = jnp.sum(e * e, axis=1, keepdims=True) - rad * rad
    h_in = jnp.dot(h, imap_ref[...], preferred_element_type=jnp.float32)
    c_in = jnp.dot(cpre_ref[...], winT_ref[...],
                   preferred_element_type=jnp.float32) + b_ref[...]
    f = h_in - c_in
    di_ref[...] = jnp.sum(f * f, axis=1, keepdims=True)


def _tc_scores(h_g, t_g, c_g, rad_g, ridx, cpre_g, rel_pad, imap, win_t,
               b2d):
    grid = _B // _TB
    row_spec = pl.BlockSpec((_TB, _D), lambda i: (i, 0))
    full = lambda shape: pl.BlockSpec(shape, lambda i: (0, 0))
    return pl.pallas_call(
        _tc_body,
        grid=(grid,),
        in_specs=[
            row_spec, row_spec, row_spec,
            pl.BlockSpec((_TB, 1), lambda i: (i, 0)),
            pl.BlockSpec((_TB, 1), lambda i: (i, 0)),
            pl.BlockSpec((_TB, _PRE), lambda i: (i, 0)),
            full((128, _D)), full((_D, _D)), full((_PRE, _D)), full((1, _D)),
        ],
        out_specs=[
            pl.BlockSpec((_TB, 1), lambda i: (i, 0)),
            pl.BlockSpec((_TB, 1), lambda i: (i, 0)),
            pl.BlockSpec((_TB, 1), lambda i: (i, 0)),
        ],
        out_shape=[
            jax.ShapeDtypeStruct((_B, 1), jnp.float32),
            jax.ShapeDtypeStruct((_B, 1), jnp.float32),
            jax.ShapeDtypeStruct((_B, 1), jnp.float32),
        ],
    )(h_g, t_g, c_g, rad_g, ridx, cpre_g, rel_pad, imap, win_t, b2d)


def kernel(h_idx, r_idx, t_idx, c_idx, instance_vec_ex, relation_vec,
           concept_vec_ex, concept_r, concept_vec_in, W_in, b_in,
           instance_map):
    h_g, t_g, c_g, rad_g, cpre_g = _sc_gather(
        h_idx, t_idx, c_idx,
        instance_vec_ex, concept_vec_ex, concept_r, concept_vec_in)
    ridx = r_idx.reshape(_B, 1)
    rel_pad = jnp.zeros((128, _D), jnp.float32).at[:relation_vec.shape[0]].set(
        relation_vec)
    st, de, di = _tc_scores(h_g, t_g, c_g, rad_g, ridx, cpre_g,
                            rel_pad, instance_map, W_in.T,
                            b_in.reshape(1, _D))
    return jnp.concatenate([st, de, di], axis=1)


# speedup vs baseline: 5.0002x; 5.0002x over previous
import functools

import jax
import jax.numpy as jnp
from jax import lax
from jax.experimental import pallas as pl
from jax.experimental.pallas import tpu as pltpu
from jax.experimental.pallas import tpu_sc as plsc

_B = 16384
_D = 64
_PRE = 384
_NC, _NS = 2, 16
_NW = _NC * _NS
_BPW = _B // _NW
_C = 64
_NCHUNK = _BPW // _C
_G = 4
_SG = _C // _G

_MESH = plsc.VectorSubcoreMesh(core_axis_name="c", subcore_axis_name="s",
                               num_cores=_NC, num_subcores=_NS)


def _k1_body(c_idx, cpre_tab, cpre_out, ci_v, cpre_v, gsem0, gsem1, wsem):
    wid = lax.axis_index("s") * _NC + lax.axis_index("c")
    base = wid * _BPW
    pltpu.sync_copy(c_idx.at[pl.ds(base, _BPW)], ci_v)
    gsems = (gsem0, gsem1)

    def fire_gathers(k):
        s = k % 2
        return [pltpu.async_copy(
            cpre_tab.at[ci_v.at[pl.ds(k * _C + g * _SG, _SG)]],
            cpre_v.at[s].at[pl.ds(g * _SG, _SG)],
            gsems[s]) for g in range(_G)]

    gcps = {0: fire_gathers(0), 1: fire_gathers(1)}
    for k in range(_NCHUNK):
        for cp in gcps.pop(k):
            cp.wait()
        pltpu.async_copy(cpre_v.at[k % 2],
                         cpre_out.at[pl.ds(base + k * _C, _C)], wsem).wait()
        if k + 2 < _NCHUNK:
            gcps[k + 2] = fire_gathers(k + 2)


_k1 = functools.partial(
    pl.kernel,
    out_type=jax.ShapeDtypeStruct((_B, _PRE), jnp.float32),
    mesh=_MESH,
    compiler_params=pltpu.CompilerParams(use_tc_tiling_on_sc=True),
    scratch_types=[
        pltpu.VMEM((_BPW,), jnp.int32),
        pltpu.VMEM((2, _C, _PRE), jnp.float32),
        pltpu.SemaphoreType.DMA,
        pltpu.SemaphoreType.DMA,
        pltpu.SemaphoreType.DMA,
    ],
)(_k1_body)


def _k2_body(h_idx, t_idx, c_idx,
             inst_tab, conc_tab, rad16_tab,
             h_out, t_out, c_out, rad16_out,
             hi_v, ti_v, ci_v, cd_v,
             h_v, t_v, c_v, rad16_v,
             gsem0, gsem1, wsem0, wsem1):
    wid = lax.axis_index("s") * _NC + lax.axis_index("c")
    base = wid * _BPW
    four = jnp.full((16,), 4, jnp.int32)
    pltpu.sync_copy(h_idx.at[pl.ds(base, _BPW)], hi_v)
    pltpu.sync_copy(t_idx.at[pl.ds(base, _BPW)], ti_v)
    pltpu.sync_copy(c_idx.at[pl.ds(base, _BPW)], ci_v)
    for j in range(_BPW // 16):
        sl = pl.ds(j * 16, 16)
        cd_v[sl] = lax.shift_right_logical(ci_v[sl], four)

    gsems = (gsem0, gsem1)
    wsems = (wsem0, wsem1)
    tables = ((inst_tab, hi_v, h_v, h_out),
              (inst_tab, ti_v, t_v, t_out),
              (conc_tab, ci_v, c_v, c_out),
              (rad16_tab, cd_v, rad16_v, rad16_out))

    def fire_gathers(k):
        s = k % 2
        cps = []
        for tab, idx, buf, _ in tables:
            for g in range(_G):
                cps.append(pltpu.async_copy(
                    tab.at[idx.at[pl.ds(k * _C + g * _SG, _SG)]],
                    buf.at[s].at[pl.ds(g * _SG, _SG)],
                    gsems[s]))
        return cps

    def fire_writes(k):
        s = k % 2
        gb = pl.ds(base + k * _C, _C)
        return [pltpu.async_copy(buf.at[s], out.at[gb], wsems[s])
                for _, _, buf, out in tables]

    gcps = {0: fire_gathers(0), 1: fire_gathers(1)}
    for k in range(_NCHUNK):
        for cp in gcps.pop(k):
            cp.wait()
        wcps = fire_writes(k)
        for cp in wcps:
            cp.wait()
        if k + 2 < _NCHUNK:
            gcps[k + 2] = fire_gathers(k + 2)


_k2 = functools.partial(
    pl.kernel,
    out_type=(
        jax.ShapeDtypeStruct((_B, _D), jnp.float32),
        jax.ShapeDtypeStruct((_B, _D), jnp.float32),
        jax.ShapeDtypeStruct((_B, _D), jnp.float32),
        jax.ShapeDtypeStruct((_B, 16), jnp.float32),
    ),
    mesh=_MESH,
    compiler_params=pltpu.CompilerParams(use_tc_tiling_on_sc=False),
    scratch_types=[
        pltpu.VMEM((_BPW,), jnp.int32),
        pltpu.VMEM((_BPW,), jnp.int32),
        pltpu.VMEM((_BPW,), jnp.int32),
        pltpu.VMEM((_BPW,), jnp.int32),
        pltpu.VMEM((2, _C, _D), jnp.float32),
        pltpu.VMEM((2, _C, _D), jnp.float32),
        pltpu.VMEM((2, _C, _D), jnp.float32),
        pltpu.VMEM((2, _C, 16), jnp.float32),
        pltpu.SemaphoreType.DMA,
        pltpu.SemaphoreType.DMA,
        pltpu.SemaphoreType.DMA,
        pltpu.SemaphoreType.DMA,
    ],
)(_k2_body)


_TB = 1024


def _tc_body(h_ref, t_ref, c_ref, rad16_ref, cmod_ref, ridx_ref, cpre_ref,
             rel_ref, imap_ref, winT_ref, b_ref, st_ref, de_ref, di_ref):
    h = h_ref[...]
    rlane = lax.broadcasted_iota(jnp.int32, (_TB, 128), 1)
    ronehot = jnp.where(rlane == ridx_ref[...], 1.0, 0.0)
    r = jnp.dot(ronehot, rel_ref[...], preferred_element_type=jnp.float32)
    d = h + r - t_ref[...]
    st_ref[...] = jnp.sum(d * d, axis=1, keepdims=True)
    e = h - c_ref[...]
    lane = lax.broadcasted_iota(jnp.int32, (_TB, 16), 1)
    picked = jnp.where(lane == cmod_ref[...], rad16_ref[...], 0.0)
    rad = jnp.sum(picked, axis=1, keepdims=True)
    de_ref[...] = jnp.sum(e * e, axis=1, keepdims=True) - rad * rad
    h_in = jnp.dot(h, imap_ref[...], preferred_element_type=jnp.float32)
    c_in = jnp.dot(cpre_ref[...], winT_ref[...],
                   preferred_element_type=jnp.float32) + b_ref[...]
    f = h_in - c_in
    di_ref[...] = jnp.sum(f * f, axis=1, keepdims=True)


def _tc_scores(h_g, t_g, c_g, rad16_g, cmod, ridx, cpre_g, rel_pad,
               imap, win_t, b2d):
    grid = _B // _TB
    row_spec = pl.BlockSpec((_TB, _D), lambda i: (i, 0))
    full = lambda shape: pl.BlockSpec(shape, lambda i: (0, 0))
    return pl.pallas_call(
        _tc_body,
        grid=(grid,),
        in_specs=[
            row_spec, row_spec, row_spec,
            pl.BlockSpec((_TB, 16), lambda i: (i, 0)),
            pl.BlockSpec((_TB, 1), lambda i: (i, 0)),
            pl.BlockSpec((_TB, 1), lambda i: (i, 0)),
            pl.BlockSpec((_TB, _PRE), lambda i: (i, 0)),
            full((128, _D)), full((_D, _D)), full((_PRE, _D)), full((1, _D)),
        ],
        out_specs=[
            pl.BlockSpec((_TB, 1), lambda i: (i, 0)),
            pl.BlockSpec((_TB, 1), lambda i: (i, 0)),
            pl.BlockSpec((_TB, 1), lambda i: (i, 0)),
        ],
        out_shape=[
            jax.ShapeDtypeStruct((_B, 1), jnp.float32),
            jax.ShapeDtypeStruct((_B, 1), jnp.float32),
            jax.ShapeDtypeStruct((_B, 1), jnp.float32),
        ],
    )(h_g, t_g, c_g, rad16_g, cmod, ridx, cpre_g, rel_pad, imap, win_t, b2d)


def kernel(h_idx, r_idx, t_idx, c_idx, instance_vec_ex, relation_vec,
           concept_vec_ex, concept_r, concept_vec_in, W_in, b_in,
           instance_map):
    cpre_g = _k1(c_idx, concept_vec_in)
    rad16_tab = concept_r.reshape(-1, 16)
    h_g, t_g, c_g, rad16_g = _k2(h_idx, t_idx, c_idx,
                                 instance_vec_ex, concept_vec_ex, rad16_tab)
    cmod = (c_idx & 15).reshape(_B, 1)
    ridx = r_idx.reshape(_B, 1)
    rel_pad = jnp.zeros((128, _D), jnp.float32).at[:relation_vec.shape[0]].set(
        relation_vec)
    st, de, di = _tc_scores(h_g, t_g, c_g, rad16_g, cmod, ridx, cpre_g,
                            rel_pad, instance_map, W_in.T,
                            b_in.reshape(1, _D))
    return jnp.concatenate([st, de, di], axis=1)


# speedup vs baseline: 5.0399x; 1.0080x over previous
import functools

import jax
import jax.numpy as jnp
from jax import lax
from jax.experimental import pallas as pl
from jax.experimental.pallas import tpu as pltpu
from jax.experimental.pallas import tpu_sc as plsc

_B = 16384
_D = 64
_PRE = 384
_NC, _NS = 2, 16
_NW = _NC * _NS
_BPW = _B // _NW
_C = 64
_NCHUNK = _BPW // _C
_G = 4
_SG = _C // _G

_MESH = plsc.VectorSubcoreMesh(core_axis_name="c", subcore_axis_name="s",
                               num_cores=_NC, num_subcores=_NS)


def _k1_body(c_idx, cpre_tab, cpre_out, ci_v, cpre_v, gsem0, gsem1, wsem):
    wid = lax.axis_index("s") * _NC + lax.axis_index("c")
    base = wid * _BPW
    pltpu.sync_copy(c_idx.at[pl.ds(base, _BPW)], ci_v)
    gsems = (gsem0, gsem1)

    def fire_gathers(k):
        s = k % 2
        return [pltpu.async_copy(
            cpre_tab.at[ci_v.at[pl.ds(k * _C + g * _SG, _SG)]],
            cpre_v.at[s].at[pl.ds(g * _SG, _SG)],
            gsems[s]) for g in range(_G)]

    gcps = {0: fire_gathers(0), 1: fire_gathers(1)}
    for k in range(_NCHUNK):
        for cp in gcps.pop(k):
            cp.wait()
        pltpu.async_copy(cpre_v.at[k % 2],
                         cpre_out.at[pl.ds(base + k * _C, _C)], wsem).wait()
        if k + 2 < _NCHUNK:
            gcps[k + 2] = fire_gathers(k + 2)


_k1 = functools.partial(
    pl.kernel,
    out_type=jax.ShapeDtypeStruct((_B, _PRE), jnp.float32),
    mesh=_MESH,
    compiler_params=pltpu.CompilerParams(use_tc_tiling_on_sc=True),
    scratch_types=[
        pltpu.VMEM((_BPW,), jnp.int32),
        pltpu.VMEM((2, _C, _PRE), jnp.float32),
        pltpu.SemaphoreType.DMA,
        pltpu.SemaphoreType.DMA,
        pltpu.SemaphoreType.DMA,
    ],
)(_k1_body)


def _k2_body(h_idx, t_idx, c_idx,
             inst_tab, conc_tab, rad16_tab,
             h_out, t_out, c_out, rad16_out,
             hi_v, ti_v, ci_v, cd_v,
             h_v, t_v, c_v, rad16_v,
             gsem0, gsem1, wsem0, wsem1):
    wid = lax.axis_index("s") * _NC + lax.axis_index("c")
    base = wid * _BPW
    four = jnp.full((16,), 4, jnp.int32)
    pltpu.sync_copy(h_idx.at[pl.ds(base, _BPW)], hi_v)
    pltpu.sync_copy(t_idx.at[pl.ds(base, _BPW)], ti_v)
    pltpu.sync_copy(c_idx.at[pl.ds(base, _BPW)], ci_v)
    for j in range(_BPW // 16):
        sl = pl.ds(j * 16, 16)
        cd_v[sl] = lax.shift_right_logical(ci_v[sl], four)

    gsems = (gsem0, gsem1)
    wsems = (wsem0, wsem1)
    tables = ((inst_tab, hi_v, h_v, h_out),
              (inst_tab, ti_v, t_v, t_out),
              (conc_tab, ci_v, c_v, c_out),
              (rad16_tab, cd_v, rad16_v, rad16_out))

    def fire_gathers(k):
        s = k % 2
        cps = []
        for tab, idx, buf, _ in tables:
            for g in range(_G):
                cps.append(pltpu.async_copy(
                    tab.at[idx.at[pl.ds(k * _C + g * _SG, _SG)]],
                    buf.at[s].at[pl.ds(g * _SG, _SG)],
                    gsems[s]))
        return cps

    def fire_writes(k):
        s = k % 2
        gb = pl.ds(base + k * _C, _C)
        return [pltpu.async_copy(buf.at[s], out.at[gb], wsems[s])
                for _, _, buf, out in tables]

    gcps = {0: fire_gathers(0), 1: fire_gathers(1)}
    for k in range(_NCHUNK):
        for cp in gcps.pop(k):
            cp.wait()
        wcps = fire_writes(k)
        for cp in wcps:
            cp.wait()
        if k + 2 < _NCHUNK:
            gcps[k + 2] = fire_gathers(k + 2)


_k2 = functools.partial(
    pl.kernel,
    out_type=(
        jax.ShapeDtypeStruct((_B, _D), jnp.float32),
        jax.ShapeDtypeStruct((_B, _D), jnp.float32),
        jax.ShapeDtypeStruct((_B, _D), jnp.float32),
        jax.ShapeDtypeStruct((_B, 16), jnp.float32),
    ),
    mesh=_MESH,
    compiler_params=pltpu.CompilerParams(use_tc_tiling_on_sc=False),
    scratch_types=[
        pltpu.VMEM((_BPW,), jnp.int32),
        pltpu.VMEM((_BPW,), jnp.int32),
        pltpu.VMEM((_BPW,), jnp.int32),
        pltpu.VMEM((_BPW,), jnp.int32),
        pltpu.VMEM((2, _C, _D), jnp.float32),
        pltpu.VMEM((2, _C, _D), jnp.float32),
        pltpu.VMEM((2, _C, _D), jnp.float32),
        pltpu.VMEM((2, _C, 16), jnp.float32),
        pltpu.SemaphoreType.DMA,
        pltpu.SemaphoreType.DMA,
        pltpu.SemaphoreType.DMA,
        pltpu.SemaphoreType.DMA,
    ],
)(_k2_body)


_TB = 2048


def _tc_body(h_ref, t_ref, c_ref, rad16_ref, cmod_ref, ridx_ref, cpre_ref,
             rel_ref, imap_ref, winT_ref, b_ref, st_ref, de_ref, di_ref):
    h = h_ref[...]
    rlane = lax.broadcasted_iota(jnp.int32, (_TB, 128), 1)
    ronehot = jnp.where(rlane == ridx_ref[...], 1.0, 0.0)
    r = jnp.dot(ronehot, rel_ref[...], preferred_element_type=jnp.float32)
    d = h + r - t_ref[...]
    st_ref[...] = jnp.sum(d * d, axis=1, keepdims=True)
    e = h - c_ref[...]
    lane = lax.broadcasted_iota(jnp.int32, (_TB, 16), 1)
    picked = jnp.where(lane == cmod_ref[...], rad16_ref[...], 0.0)
    rad = jnp.sum(picked, axis=1, keepdims=True)
    de_ref[...] = jnp.sum(e * e, axis=1, keepdims=True) - rad * rad
    h_in = jnp.dot(h, imap_ref[...], preferred_element_type=jnp.float32)
    c_in = jnp.dot(cpre_ref[...], winT_ref[...],
                   preferred_element_type=jnp.float32) + b_ref[...]
    f = h_in - c_in
    di_ref[...] = jnp.sum(f * f, axis=1, keepdims=True)


def _tc_scores(h_g, t_g, c_g, rad16_g, cmod, ridx, cpre_g, rel_pad,
               imap, win_t, b2d):
    grid = _B // _TB
    row_spec = pl.BlockSpec((_TB, _D), lambda i: (i, 0))
    full = lambda shape: pl.BlockSpec(shape, lambda i: (0, 0))
    return pl.pallas_call(
        _tc_body,
        grid=(grid,),
        in_specs=[
            row_spec, row_spec, row_spec,
            pl.BlockSpec((_TB, 16), lambda i: (i, 0)),
            pl.BlockSpec((_TB, 1), lambda i: (i, 0)),
            pl.BlockSpec((_TB, 1), lambda i: (i, 0)),
            pl.BlockSpec((_TB, _PRE), lambda i: (i, 0)),
            full((128, _D)), full((_D, _D)), full((_PRE, _D)), full((1, _D)),
        ],
        out_specs=[
            pl.BlockSpec((_TB, 1), lambda i: (i, 0)),
            pl.BlockSpec((_TB, 1), lambda i: (i, 0)),
            pl.BlockSpec((_TB, 1), lambda i: (i, 0)),
        ],
        out_shape=[
            jax.ShapeDtypeStruct((_B, 1), jnp.float32),
            jax.ShapeDtypeStruct((_B, 1), jnp.float32),
            jax.ShapeDtypeStruct((_B, 1), jnp.float32),
        ],
    )(h_g, t_g, c_g, rad16_g, cmod, ridx, cpre_g, rel_pad, imap, win_t, b2d)


def kernel(h_idx, r_idx, t_idx, c_idx, instance_vec_ex, relation_vec,
           concept_vec_ex, concept_r, concept_vec_in, W_in, b_in,
           instance_map):
    cpre_g = _k1(c_idx, concept_vec_in)
    rad16_tab = concept_r.reshape(-1, 16)
    h_g, t_g, c_g, rad16_g = _k2(h_idx, t_idx, c_idx,
                                 instance_vec_ex, concept_vec_ex, rad16_tab)
    cmod = (c_idx & 15).reshape(_B, 1)
    ridx = r_idx.reshape(_B, 1)
    rel_pad = jnp.zeros((128, _D), jnp.float32).at[:relation_vec.shape[0]].set(
        relation_vec)
    st, de, di = _tc_scores(h_g, t_g, c_g, rad16_g, cmod, ridx, cpre_g,
                            rel_pad, instance_map, W_in.T,
                            b_in.reshape(1, _D))
    return jnp.concatenate([st, de, di], axis=1)
